# baseline (speedup 1.0000x reference)
import functools

import jax
import jax.numpy as jnp
from jax import lax
from jax.experimental import pallas as pl
from jax.experimental.pallas import tpu as pltpu
from jax.experimental.pallas import tpu_sc as plsc

N = 10000
E = 160000
D_IN = 256
DIM = 512
NUM_CLASSES = 64

NC = 2
NS = 16
CHUNK = 128
NCH = 80
HALF = NCH // 2
EPAD = NS * NCH * CHUNK
PAD_DST = 10104
NPAD = 10112
ROWS_PER_TILE = NPAD // NS

NHALF = 5056
ZROWS = 10176
ACCR = 5120
GARB = NHALF
DUMMY = 20000
EPT = NS * NCH * CHUNK // NS
CAP = EPT + 128
NCHK = CAP // 64
HCHK = NCHK // 2
SEG = HCHK * 64

BN = 400
GRID = N // BN

_MESH = dict(core_axis_name="c", subcore_axis_name="s",
             num_cores=NC, num_subcores=NS)


def _dot(a, b):
    return lax.dot_general(a, b, (((1,), (0,)), ((), ())),
                           precision=lax.Precision.DEFAULT,
                           preferred_element_type=jnp.float32)


def _fill(ref, rows, value):
    def body(i, carry):
        for v in range(8):
            ref[i, pl.ds(v * 16, 16)] = jnp.full((16,), value, jnp.float32)
        return carry
    lax.fori_loop(0, rows, body, 0)


@functools.cache
def _get_deg_kernel():
    return functools.partial(
        pl.kernel,
        out_type=[jax.ShapeDtypeStruct((NPAD, 128), jnp.float32)
                  for _ in range(NC)],
        mesh=plsc.VectorSubcoreMesh(**_MESH),
        scratch_types=[
            pltpu.VMEM_SHARED((NPAD, 128), jnp.float32),
            pltpu.VMEM((NCH, CHUNK), jnp.int32),
            pltpu.VMEM((CHUNK, 128), jnp.float32),
            pltpu.VMEM((CHUNK, 128), jnp.float32),
        ],
    )(_deg_body)


def _deg_body(dst_hbm, deg0_out, deg1_out, acc, dstv, ones_v, zer_v):
    c = lax.axis_index("c")
    s = lax.axis_index("s")

    _fill(ones_v, CHUNK, 1.0)
    _fill(zer_v, CHUNK, 0.0)

    base = s * ROWS_PER_TILE
    for r in range(4):
        pltpu.sync_copy(zer_v, acc.at[pl.ds(base + r * CHUNK, CHUNK)])
    pltpu.sync_copy(zer_v.at[pl.ds(0, 120)],
                    acc.at[pl.ds(base + 4 * CHUNK, 120)])
    pltpu.sync_copy(dst_hbm.at[s], dstv)
    plsc.subcore_barrier()

    def step(j, carry):
        pltpu.sync_copy(ones_v, acc.at[dstv.at[j]], add=True)
        return carry

    lax.fori_loop(c * HALF, (c + 1) * HALF, step, 0)
    plsc.subcore_barrier()

    sl = pl.ds(base, ROWS_PER_TILE)

    @pl.when(c == 0)
    def _():
        pltpu.sync_copy(acc.at[sl], deg0_out.at[sl])

    @pl.when(c == 1)
    def _():
        pltpu.sync_copy(acc.at[sl], deg1_out.at[sl])


@functools.cache
def _get_part_kernel():
    return functools.partial(
        pl.kernel,
        out_type=[jax.ShapeDtypeStruct((NS, CAP), jnp.int32),
                  jax.ShapeDtypeStruct((NS, CAP), jnp.int32),
                  jax.ShapeDtypeStruct((NS, 128), jnp.int32)],
        mesh=plsc.VectorSubcoreMesh(**_MESH),
        scratch_types=[
            pltpu.VMEM((NCH, CHUNK), jnp.int32),
            pltpu.VMEM((NCH, CHUNK), jnp.int32),
            pltpu.VMEM((CAP + 16,), jnp.int32),
            pltpu.VMEM((CAP + 16,), jnp.int32),
            pltpu.VMEM((128,), jnp.int32),
        ],
    )(_part_body)


def _psum16(w):
    i16 = lax.iota(jnp.int32, 16)
    dn = lax.GatherDimensionNumbers(offset_dims=(), collapsed_slice_dims=(0,),
                                    start_index_map=(0,))
    zero = jnp.zeros((16,), jnp.int32)
    for sh in (1, 2, 4, 8):
        t = lax.gather(w, jnp.maximum(i16 - sh, 0)[:, None], dn, (1,),
                       mode=lax.GatherScatterMode.PROMISE_IN_BOUNDS)
        w = w + jnp.where(i16 >= sh, t, zero)
    return w


def _hsum16(w):
    i16 = lax.iota(jnp.int32, 16)
    dn = lax.GatherDimensionNumbers(offset_dims=(), collapsed_slice_dims=(0,),
                                    start_index_map=(0,))
    for sh in (8, 4, 2, 1):
        w = w + lax.gather(w, (i16 ^ sh)[:, None], dn, (1,),
                           mode=lax.GatherScatterMode.PROMISE_IN_BOUNDS)
    return w[0]


def _part_body(src_hbm, dst_hbm, srcb_out, dstb_out, cnt_out,
               srcv, dstv, srcb, dstb, cntv):
    c = lax.axis_index("c")
    s = lax.axis_index("s")

    @pl.when(c == 0)
    def _():
        pltpu.sync_copy(src_hbm.at[s], srcv)
        pltpu.sync_copy(dst_hbm.at[s], dstv)

        def row(j, carry):
            p0, p1 = carry
            for g in range(8):
                d = dstv[j, pl.ds(16 * g, 16)]
                sg = srcv[j, pl.ds(16 * g, 16)]
                m = d < NHALF
                w = lax.shift_right_logical(d - NHALF, 31)
                pre_inc = _psum16(w)
                nlo = pre_inc[15]
                nhi = 16 - nlo
                pre = pre_inc - w
                i16 = lax.iota(jnp.int32, 16)
                prehi = i16 - pre
                pos = jnp.where(m, p0 + pre, p1 - nhi + prehi)
                plsc.store_scatter(srcb, [pos], sg)
                plsc.store_scatter(dstb, [pos], d)
                p0 = p0 + nlo
                p1 = p1 - nhi
            return p0, p1

        p0, p1 = lax.fori_loop(0, NCH, row, (0, CAP))
        for t in range(8):
            srcb[pl.ds(p0 + 16 * t, 16)] = jnp.zeros((16,), jnp.int32)
            dstb[pl.ds(p0 + 16 * t, 16)] = jnp.full((16,), DUMMY, jnp.int32)
        cntv[pl.ds(0, 16)] = jnp.full((16,), p0, jnp.int32)
        pltpu.sync_copy(srcb.at[pl.ds(0, CAP)], srcb_out.at[s])
        pltpu.sync_copy(dstb.at[pl.ds(0, CAP)], dstb_out.at[s])
        pltpu.sync_copy(cntv, cnt_out.at[s])


def _iota16():
    return lax.iota(jnp.int32, 16)


def _row_step(j, r_base, z_ref, srcbv, dstbv, acc, buf, sem0, sem1):
    jl = j - r_base
    g0 = pltpu.make_async_copy(z_ref.at[srcbv.at[jl, pl.ds(0, 64)]],
                               buf.at[pl.ds(0, 64)], sem0)
    g1 = pltpu.make_async_copy(z_ref.at[srcbv.at[jl, pl.ds(64, 64)]],
                               buf.at[pl.ds(64, 64)], sem1)
    g0.start()
    g1.start()
    g0.wait()
    g1.wait()
    pltpu.sync_copy(buf, acc.at[dstbv.at[jl]], add=True)


def _agg_segment(z_ref, srcbv, dstbv, acc, buf, sem0, sem1, r0, r1, r_base):
    def step(j, carry):
        _row_step(j, r_base, z_ref, srcbv, dstbv, acc, buf, sem0, sem1)
        return carry

    lax.fori_loop(r0, r1, step, 0)


def _agg_pass(s, z_ref, out_ref, srcb_hbm, idx_hbm, acc, srcbv, dstbv,
              buf, sem0, sem1, c0, c1, off):
    init_sl = pl.ds(off + 320 * s, 320)
    acc_sl = pl.ds(320 * s, 320)
    pltpu.sync_copy(z_ref.at[init_sl], acc.at[acc_sl])
    plsc.subcore_barrier()

    for rs, nr in ((0, 40), (40, 41)):
        r0 = jnp.maximum(c0, rs)
        r1 = jnp.minimum(c1, rs + nr)

        @pl.when(r1 > r0)
        def _():
            pltpu.sync_copy(srcb_hbm.at[s, pl.ds(rs, nr)],
                            srcbv.at[pl.ds(0, nr)])
            pltpu.sync_copy(idx_hbm.at[s, pl.ds(rs, nr)],
                            dstbv.at[pl.ds(0, nr)])
            _agg_segment(z_ref, srcbv, dstbv, acc, buf, sem0, sem1,
                         r0, r1, rs)

    plsc.subcore_barrier()
    pltpu.sync_copy(acc.at[acc_sl], out_ref.at[init_sl])
    plsc.subcore_barrier()


@functools.cache
def _get_agg_kernel():
    return functools.partial(
        pl.kernel,
        out_type=[jax.ShapeDtypeStruct((ZROWS, 256), jnp.float32)
                  for _ in range(NC)],
        mesh=plsc.VectorSubcoreMesh(**_MESH),
        compiler_params=pltpu.CompilerParams(use_tc_tiling_on_sc=False),
        scratch_types=[
            pltpu.VMEM_SHARED((ACCR, 256), jnp.float32),
            pltpu.VMEM((41, 128), jnp.int32),
            pltpu.VMEM((41, 128), jnp.int32),
            pltpu.VMEM((1, 128), jnp.int32),
            pltpu.VMEM((128, 256), jnp.float32),
            pltpu.SemaphoreType.DMA,
            pltpu.SemaphoreType.DMA,
        ],
    )(_agg_body)


def _agg_body(zA, zB, srcb_hbm, idxlo_hbm, idxhi_hbm, cnt_hbm, oA, oB,
              acc, srcbv, dstbv, cntv, buf, sem0, sem1):
    c = lax.axis_index("c")
    s = lax.axis_index("s")
    pltpu.sync_copy(cnt_hbm.at[s], cntv)
    p0 = cntv[0, pl.ds(0, 16)][0]
    n0ch = lax.shift_right_logical(p0 + 63, 6)

    r_lo_end = lax.shift_right_logical(n0ch + 1, 1)
    r_hi_start = lax.shift_right_logical(n0ch, 1)

    def run(z_ref, out_ref):
        _agg_pass(s, z_ref, out_ref, srcb_hbm, idxlo_hbm, acc, srcbv, dstbv,
                  buf, sem0, sem1, 0, r_lo_end, 0)
        _agg_pass(s, z_ref, out_ref, srcb_hbm, idxhi_hbm, acc, srcbv, dstbv,
                  buf, sem0, sem1, r_hi_start, CAP // 128, NHALF)

    @pl.when(c == 0)
    def _():
        run(zA, oA)

    @pl.when(c == 1)
    def _():
        run(zB, oB)


def _dinv_of(d0_blk, d1_blk):
    return lax.rsqrt(d0_blk[:, 0:1] + d1_blk[:, 0:1] + 1.0)


def _tc1a_body(x_ref, w_ref, zA, zB):
    y = _dot(x_ref[...], w_ref[...])
    zA[...] = y[:, 0:256]
    zB[...] = y[:, 256:512]


def _tc1b_body(yA, yB, d0_ref, d1_ref, zA, zB):
    dinv = _dinv_of(d0_ref[...], d1_ref[...])
    zA[...] = yA[...] * dinv
    zB[...] = yB[...] * dinv


def _tc_mid_body(aA, aB, d0_ref, d1_ref, b_ref, w_ref, x_out, zA, zB):
    dinv = _dinv_of(d0_ref[...], d1_ref[...])
    agg = jnp.concatenate([aA[...], aB[...]], axis=1)
    xl = jnp.maximum(agg * dinv + b_ref[...], 0.0)
    x_out[...] = xl
    z = _dot(xl, w_ref[...]) * dinv
    zA[...] = z[:, 0:256]
    zB[...] = z[:, 256:512]


def _tc_final_body(aA, aB, d0_ref, d1_ref, b_ref, x1_ref, x2_ref,
                   wl1_ref, wl2_ref, wl3_ref, bl_ref, out_ref):
    dinv = _dinv_of(d0_ref[...], d1_ref[...])
    agg = jnp.concatenate([aA[...], aB[...]], axis=1)
    x3 = agg * dinv + b_ref[...]
    logits = (_dot(x1_ref[...], wl1_ref[...]) +
              _dot(x2_ref[...], wl2_ref[...]) +
              _dot(x3, wl3_ref[...]) + bl_ref[...])
    m = jnp.max(logits, axis=1, keepdims=True)
    lse = jnp.log(jnp.sum(jnp.exp(logits - m), axis=1, keepdims=True)) + m
    out_ref[...] = logits - lse


def _row_spec(bm, bn):
    return pl.BlockSpec((bm, bn), lambda i: (i, 0))


def _full_spec(shape):
    return pl.BlockSpec(shape, lambda i: tuple(0 for _ in shape))


_Z_OUT = [jax.ShapeDtypeStruct((ZROWS, 256), jnp.float32) for _ in range(2)]
_DEG_SPECS = [_row_spec(BN, 128), _row_spec(BN, 128)]
_Z_SPECS = [_row_spec(BN, 256), _row_spec(BN, 256)]


def _tc1a(x, w1):
    return pl.pallas_call(
        _tc1a_body,
        grid=(GRID,),
        in_specs=[_row_spec(BN, D_IN), _full_spec((D_IN, DIM))],
        out_specs=_Z_SPECS,
        out_shape=_Z_OUT,
    )(x, w1)


def _tc1b(y, deg):
    return pl.pallas_call(
        _tc1b_body,
        grid=(GRID,),
        in_specs=_Z_SPECS + _DEG_SPECS,
        out_specs=_Z_SPECS,
        out_shape=_Z_OUT,
    )(*y, *deg)


def _tc_mid(a, deg, b, w):
    return pl.pallas_call(
        _tc_mid_body,
        grid=(GRID,),
        in_specs=_Z_SPECS + _DEG_SPECS + [
            _full_spec((1, DIM)), _full_spec((DIM, DIM))],
        out_specs=[_row_spec(BN, DIM)] + _Z_SPECS,
        out_shape=[jax.ShapeDtypeStruct((N, DIM), jnp.float32)] + _Z_OUT,
    )(*a, *deg, b.reshape(1, DIM), w)


def _tc_final(a, deg, b3, x1, x2, wl, bl):
    return pl.pallas_call(
        _tc_final_body,
        grid=(GRID,),
        in_specs=_Z_SPECS + _DEG_SPECS + [
            _full_spec((1, DIM)),
            _row_spec(BN, DIM), _row_spec(BN, DIM),
            _full_spec((DIM, NUM_CLASSES)), _full_spec((DIM, NUM_CLASSES)),
            _full_spec((DIM, NUM_CLASSES)), _full_spec((1, NUM_CLASSES))],
        out_specs=_row_spec(BN, NUM_CLASSES),
        out_shape=jax.ShapeDtypeStruct((N, NUM_CLASSES), jnp.float32),
    )(*a, *deg, b3.reshape(1, DIM), x1, x2,
      wl[0:DIM], wl[DIM:2 * DIM], wl[2 * DIM:3 * DIM],
      bl.reshape(1, NUM_CLASSES))


def kernel(x, edge_index, W1, b1, W2, b2, W3, b3, Wl, bl):
    src = edge_index[0]
    dst = edge_index[1]
    pad = EPAD - E
    src_p = jnp.concatenate(
        [src, jnp.zeros((pad,), jnp.int32)]).reshape(NS, NCH, CHUNK)
    dst_p = jnp.concatenate(
        [dst, jnp.full((pad,), PAD_DST, jnp.int32)]).reshape(NS, NCH, CHUNK)

    deg = _get_deg_kernel()(dst_p)
    src_t = src_p.reshape(NS, EPT)
    dst_t = dst_p.reshape(NS, EPT)
    lo = dst_t < NHALF
    pre = jnp.cumsum(lo.astype(jnp.int32), axis=1)
    nlo = pre[:, -1]
    pos_hi = CAP - jnp.cumsum((~lo).astype(jnp.int32), axis=1)
    pos = jnp.where(lo, pre - 1, pos_hi)
    rows = jnp.broadcast_to(jnp.arange(NS, dtype=jnp.int32)[:, None],
                            (NS, EPT))
    srcb = jnp.zeros((NS, CAP), jnp.int32).at[rows, pos].set(
        src_t, unique_indices=True, mode="promise_in_bounds")
    dstb = jnp.full((NS, CAP), DUMMY, jnp.int32).at[rows, pos].set(
        dst_t, unique_indices=True, mode="promise_in_bounds")
    cnt = jnp.broadcast_to(nlo[:, None], (NS, 128)).astype(jnp.int32)
    spread = jnp.arange(CAP, dtype=jnp.int32) & 7
    idx_lo = jnp.where(dstb < NHALF, dstb, GARB + spread)
    idx_hi = jnp.where((dstb >= NHALF) & (dstb < NPAD), dstb - NHALF,
                       GARB + spread)
    part = (srcb.reshape(NS, CAP // 128, 128),
            idx_lo.reshape(NS, CAP // 128, 128),
            idx_hi.reshape(NS, CAP // 128, 128),
            cnt.reshape(NS, 1, 128))
    agg_kernel = _get_agg_kernel()
    y1 = _tc1a(x, W1)
    z1 = _tc1b(y1, deg)
    a1 = agg_kernel(*z1, *part)
    x1, *z2 = _tc_mid(a1, deg, b1, W2)
    a2 = agg_kernel(*z2, *part)
    x2, *z3 = _tc_mid(a2, deg, b2, W3)
    a3 = agg_kernel(*z3, *part)
    return _tc_final(a3, deg, b3, x1, x2, Wl, bl)


# speedup vs baseline: 1.8629x; 1.8629x over previous
import functools

import jax
import jax.numpy as jnp
from jax import lax
from jax.experimental import pallas as pl
from jax.experimental.pallas import tpu as pltpu
from jax.experimental.pallas import tpu_sc as plsc

N = 10000
E = 160000
D_IN = 256
DIM = 512
NUM_CLASSES = 64

NC = 2
NS = 16
CHUNK = 128
NCH = 80
HALF = NCH // 2
EPAD = NS * NCH * CHUNK
PAD_DST = 10104
NPAD = 10112
ACC_ROWS = NPAD
ROWS_PER_TILE = NPAD // NS

FB = 4
BN = 400
GRID = N // BN

_MESH = dict(core_axis_name="c", subcore_axis_name="s",
             num_cores=NC, num_subcores=NS)


def _dot(a, b):
    return lax.dot_general(a, b, (((1,), (0,)), ((), ())),
                           precision=lax.Precision.DEFAULT,
                           preferred_element_type=jnp.float32)


def _fill(ref, rows, value):
    def body(i, carry):
        for v in range(8):
            ref[i, pl.ds(v * 16, 16)] = jnp.full((16,), value, jnp.float32)
        return carry
    lax.fori_loop(0, rows, body, 0)


@functools.cache
def _get_deg_kernel():
    return functools.partial(
        pl.kernel,
        out_type=[jax.ShapeDtypeStruct((NPAD, 128), jnp.float32)
                  for _ in range(NC)],
        mesh=plsc.VectorSubcoreMesh(**_MESH),
        compiler_params=pltpu.CompilerParams(use_tc_tiling_on_sc=False),
        scratch_types=[
            pltpu.VMEM_SHARED((ACC_ROWS, 128), jnp.float32),
            pltpu.VMEM((NCH, CHUNK), jnp.int32),
            pltpu.VMEM((CHUNK, 128), jnp.float32),
            pltpu.VMEM((CHUNK, 128), jnp.float32),
        ],
    )(_deg_body)


def _deg_body(dst_hbm, deg0_out, deg1_out, acc, dstv, ones_v, zer_v):
    c = lax.axis_index("c")
    s = lax.axis_index("s")

    _fill(ones_v, CHUNK, 1.0)
    _fill(zer_v, CHUNK, 0.0)

    base = s * ROWS_PER_TILE
    for r in range(4):
        pltpu.sync_copy(zer_v, acc.at[pl.ds(base + r * CHUNK, CHUNK)])
    pltpu.sync_copy(zer_v.at[pl.ds(0, 120)],
                    acc.at[pl.ds(base + 4 * CHUNK, 120)])
    pltpu.sync_copy(dst_hbm.at[s], dstv)
    plsc.subcore_barrier()

    def step(j, carry):
        pltpu.sync_copy(ones_v, acc.at[dstv.at[j]], add=True)
        return carry

    lax.fori_loop(c * HALF, (c + 1) * HALF, step, 0)
    plsc.subcore_barrier()

    sl = pl.ds(base, ROWS_PER_TILE)

    @pl.when(c == 0)
    def _():
        pltpu.sync_copy(acc.at[sl], deg0_out.at[sl])

    @pl.when(c == 1)
    def _():
        pltpu.sync_copy(acc.at[sl], deg1_out.at[sl])


def _agg_half(z_ref, acc, srcv, dstv, buf0, buf1, sem0, sem1):
    pltpu.make_async_copy(z_ref.at[srcv.at[0]], buf0, sem0).start()
    pltpu.make_async_copy(z_ref.at[srcv.at[1]], buf1, sem1).start()
    nk = HALF // 2

    def step(k, carry):
        j0 = 2 * k
        pltpu.make_async_copy(z_ref.at[srcv.at[j0]], buf0, sem0).wait()
        pltpu.sync_copy(buf0, acc.at[dstv.at[j0]], add=True)

        @pl.when(k < nk - 1)
        def _():
            pltpu.make_async_copy(z_ref.at[srcv.at[j0 + 2]], buf0,
                                  sem0).start()

        pltpu.make_async_copy(z_ref.at[srcv.at[j0 + 1]], buf1, sem1).wait()
        pltpu.sync_copy(buf1, acc.at[dstv.at[j0 + 1]], add=True)

        @pl.when(k < nk - 1)
        def _():
            pltpu.make_async_copy(z_ref.at[srcv.at[j0 + 3]], buf1,
                                  sem1).start()

        return carry

    lax.fori_loop(0, nk, step, 0)


def _agg_process(s, z_ref, out_ref, src_hbm, dst_hbm,
                 acc, srcv, dstv, buf0, buf1, sem0, sem1):
    init_sl = pl.ds(s * ROWS_PER_TILE, ROWS_PER_TILE)
    pltpu.sync_copy(z_ref.at[init_sl], acc.at[init_sl])
    plsc.subcore_barrier()

    for h in range(2):
        pltpu.sync_copy(src_hbm.at[s, pl.ds(h * HALF, HALF)], srcv)
        pltpu.sync_copy(dst_hbm.at[s, pl.ds(h * HALF, HALF)], dstv)
        _agg_half(z_ref, acc, srcv, dstv, buf0, buf1, sem0, sem1)

    plsc.subcore_barrier()
    pltpu.sync_copy(acc.at[init_sl], out_ref.at[init_sl])
    plsc.subcore_barrier()


@functools.cache
def _get_agg_kernel():
    return functools.partial(
        pl.kernel,
        out_type=[jax.ShapeDtypeStruct((NPAD, 128), jnp.float32)
                  for _ in range(FB)],
        mesh=plsc.VectorSubcoreMesh(**_MESH),
        compiler_params=pltpu.CompilerParams(use_tc_tiling_on_sc=False),
        scratch_types=[
            pltpu.VMEM_SHARED((ACC_ROWS, 128), jnp.float32),
            pltpu.VMEM((HALF, CHUNK), jnp.int32),
            pltpu.VMEM((HALF, CHUNK), jnp.int32),
            pltpu.VMEM((CHUNK, 128), jnp.float32),
            pltpu.VMEM((CHUNK, 128), jnp.float32),
            pltpu.SemaphoreType.DMA,
            pltpu.SemaphoreType.DMA,
        ],
    )(_agg_body)


def _agg_body(z0, z1, z2, z3, src_hbm, dst_hbm, o0, o1, o2, o3,
              acc, srcv, dstv, buf0, buf1, sem0, sem1):
    c = lax.axis_index("c")
    s = lax.axis_index("s")
    args = (src_hbm, dst_hbm, acc, srcv, dstv, buf0, buf1, sem0, sem1)

    @pl.when(c == 0)
    def _():
        _agg_process(s, z0, o0, *args)
        _agg_process(s, z1, o1, *args)

    @pl.when(c == 1)
    def _():
        _agg_process(s, z2, o2, *args)
        _agg_process(s, z3, o3, *args)


def _dinv_of(d0_blk, d1_blk):
    return lax.rsqrt(d0_blk[:, 0:1] + d1_blk[:, 0:1] + 1.0)


def _tc1a_body(x_ref, w_ref, y0, y1, y2, y3):
    y = _dot(x_ref[...], w_ref[...])
    for k, yr in enumerate((y0, y1, y2, y3)):
        yr[...] = y[:, k * 128:(k + 1) * 128]


def _tc1b_body(y0, y1, y2, y3, d0_ref, d1_ref, z0, z1, z2, z3):
    dinv = _dinv_of(d0_ref[...], d1_ref[...])
    for yr, zr in zip((y0, y1, y2, y3), (z0, z1, z2, z3)):
        zr[...] = yr[...] * dinv


def _tc_mid_body(a0, a1, a2, a3, d0_ref, d1_ref, b_ref, w_ref,
                 x_out, z0, z1, z2, z3):
    dinv = _dinv_of(d0_ref[...], d1_ref[...])
    agg = jnp.concatenate([a0[...], a1[...], a2[...], a3[...]], axis=1)
    xl = jnp.maximum(agg * dinv + b_ref[...], 0.0)
    x_out[...] = xl
    z = _dot(xl, w_ref[...]) * dinv
    for k, zr in enumerate((z0, z1, z2, z3)):
        zr[...] = z[:, k * 128:(k + 1) * 128]


def _tc_final_body(a0, a1, a2, a3, d0_ref, d1_ref, b_ref, x1_ref, x2_ref,
                   wl1_ref, wl2_ref, wl3_ref, bl_ref, out_ref):
    dinv = _dinv_of(d0_ref[...], d1_ref[...])
    agg = jnp.concatenate([a0[...], a1[...], a2[...], a3[...]], axis=1)
    x3 = agg * dinv + b_ref[...]
    logits = (_dot(x1_ref[...], wl1_ref[...]) +
              _dot(x2_ref[...], wl2_ref[...]) +
              _dot(x3, wl3_ref[...]) + bl_ref[...])
    m = jnp.max(logits, axis=1, keepdims=True)
    lse = jnp.log(jnp.sum(jnp.exp(logits - m), axis=1, keepdims=True)) + m
    out_ref[...] = logits - lse


def _row_spec(bm, bn):
    return pl.BlockSpec((bm, bn), lambda i: (i, 0))


def _full_spec(shape):
    return pl.BlockSpec(shape, lambda i: tuple(0 for _ in shape))


_Z_OUT = [jax.ShapeDtypeStruct((NPAD, 128), jnp.float32) for _ in range(FB)]
_DEG_SPECS = [_row_spec(BN, 128), _row_spec(BN, 128)]


def _tc1a(x, w1):
    return pl.pallas_call(
        _tc1a_body,
        grid=(GRID,),
        in_specs=[_row_spec(BN, D_IN), _full_spec((D_IN, DIM))],
        out_specs=[_row_spec(BN, 128) for _ in range(FB)],
        out_shape=_Z_OUT,
    )(x, w1)


def _tc1b(y, deg):
    return pl.pallas_call(
        _tc1b_body,
        grid=(GRID,),
        in_specs=[_row_spec(BN, 128)] * FB + _DEG_SPECS,
        out_specs=[_row_spec(BN, 128) for _ in range(FB)],
        out_shape=_Z_OUT,
    )(*y, *deg)


def _tc_mid(a, deg, b, w):
    return pl.pallas_call(
        _tc_mid_body,
        grid=(GRID,),
        in_specs=[_row_spec(BN, 128)] * FB + _DEG_SPECS + [
            _full_spec((1, DIM)), _full_spec((DIM, DIM))],
        out_specs=[_row_spec(BN, DIM)] + [_row_spec(BN, 128)] * FB,
        out_shape=[jax.ShapeDtypeStruct((N, DIM), jnp.float32)] + _Z_OUT,
    )(*a, *deg, b.reshape(1, DIM), w)


def _tc_final(a, deg, b3, x1, x2, wl, bl):
    return pl.pallas_call(
        _tc_final_body,
        grid=(GRID,),
        in_specs=[_row_spec(BN, 128)] * FB + _DEG_SPECS + [
            _full_spec((1, DIM)),
            _row_spec(BN, DIM), _row_spec(BN, DIM),
            _full_spec((DIM, NUM_CLASSES)), _full_spec((DIM, NUM_CLASSES)),
            _full_spec((DIM, NUM_CLASSES)), _full_spec((1, NUM_CLASSES))],
        out_specs=_row_spec(BN, NUM_CLASSES),
        out_shape=jax.ShapeDtypeStruct((N, NUM_CLASSES), jnp.float32),
    )(*a, *deg, b3.reshape(1, DIM), x1, x2,
      wl[0:DIM], wl[DIM:2 * DIM], wl[2 * DIM:3 * DIM],
      bl.reshape(1, NUM_CLASSES))


def kernel(x, edge_index, W1, b1, W2, b2, W3, b3, Wl, bl):
    src = edge_index[0]
    dst = edge_index[1]
    pad = EPAD - E
    src_p = jnp.concatenate(
        [src, jnp.zeros((pad,), jnp.int32)]).reshape(NS, NCH, CHUNK)
    dst_p = jnp.concatenate(
        [dst, jnp.full((pad,), PAD_DST, jnp.int32)]).reshape(NS, NCH, CHUNK)

    deg = _get_deg_kernel()(dst_p)
    agg_kernel = _get_agg_kernel()
    y1 = _tc1a(x, W1)
    z1 = _tc1b(y1, deg)
    a1 = agg_kernel(*z1, src_p, dst_p)
    x1, *z2 = _tc_mid(a1, deg, b1, W2)
    a2 = agg_kernel(*z2, src_p, dst_p)
    x2, *z3 = _tc_mid(a2, deg, b2, W3)
    a3 = agg_kernel(*z3, src_p, dst_p)
    return _tc_final(a3, deg, b3, x1, x2, Wl, bl)
